# baseline (speedup 1.0000x reference)
import functools

import jax
import jax.numpy as jnp
from jax import lax
from jax.experimental import pallas as pl
from jax.experimental.pallas import tpu as pltpu
from jax.experimental.pallas import tpu_sc as plsc

N = 50000
E = 3200000
INPUT_DIM = 128
HIDDEN = 16
OUT = 16

_BN = 200
_NW = 32
_NP = 50048
_EPW = 102400
_EP = _EPW * _NW
_CHK = 8
_CH = _CHK * 128
_NCHUNK = _EPW // _CH
_NC0 = 184
_NC1 = 2 * _NCHUNK - _NC0
_RPT = _NP // 16
_NHS = 51200
_DPT = _NHS // 16

_mesh = plsc.VectorSubcoreMesh(core_axis_name="c", subcore_axis_name="s")


def _deg_body(dstp, out, idx_v, ones_v, rid_v, buf_v, acc, sem):
    c = lax.axis_index("c")
    s = lax.axis_index("s")
    wid = s * 2 + c
    for j in range(8):
        ones_v[pl.ds(j * 16, 16)] = jnp.ones((16,), jnp.float32)
        buf_v[pl.ds(j * 16, 16)] = jnp.zeros((16,), jnp.float32)
    base = s * _DPT
    for k in range(_DPT // 128):
        for o in range(8):
            rid_v[k, pl.ds(o * 16, 16)] = (
                jnp.arange(16, dtype=jnp.int32) + (base + k * 128 + o * 16))
    zcps = [pltpu.async_copy(buf_v.at[pl.ds(0, 128)], acc.at[rid_v.at[k]], sem)
            for k in range(_DPT // 128)]
    for cp in zcps:
        cp.wait()
    plsc.subcore_barrier()

    def body(i, carry):
        row0 = wid * (_EPW // 128) + i * _CHK
        pltpu.sync_copy(dstp.at[pl.ds(row0, _CHK), :], idx_v)
        cps = [pltpu.async_copy(ones_v, acc.at[idx_v.at[j]], sem, add=True)
               for j in range(_CHK)]
        for cp in cps:
            cp.wait()
        return carry

    lax.fori_loop(0, _NCHUNK, body, 0)
    plsc.subcore_barrier()
    gcps = [pltpu.async_copy(acc.at[rid_v.at[k]],
                             buf_v.at[pl.ds(k * 128, 128)], sem)
            for k in range(_DPT // 128)]
    for cp in gcps:
        cp.wait()
    pltpu.sync_copy(buf_v, out.at[c, pl.ds(base, _DPT)])


@functools.partial(
    pl.kernel,
    out_type=jax.ShapeDtypeStruct((2, _NHS), jnp.float32),
    mesh=_mesh,
    scratch_types=[
        pltpu.VMEM((_CHK, 128), jnp.int32),
        pltpu.VMEM((128,), jnp.float32),
        pltpu.VMEM((_DPT // 128, 128), jnp.int32),
        pltpu.VMEM((_DPT,), jnp.float32),
        pltpu.VMEM_SHARED((_NHS,), jnp.float32),
        pltpu.SemaphoreType.DMA,
    ],
)
def _deg_sc(dstp, out, idx_v, ones_v, rid_v, buf_v, acc, sem):
    _deg_body(dstp, out, idx_v, ones_v, rid_v, buf_v, acc, sem)


def _layer_body(srcp, dstp, g, z16, out,
                si0, di0, r0, si1, di1, r1, acc,
                gs0, gs1, ss0, ss1):
    c = lax.axis_index("c")
    s = lax.axis_index("s")
    bufs = ((si0, di0, r0, gs0, ss0), (si1, di1, r1, gs1, ss1))
    chunk0 = jnp.where(c == 0, s * _NC0, 16 * _NC0 + s * _NC1)
    nhalf = jnp.where(c == 0, _NC0 // 2, _NC1 // 2)

    def load_idx(buf, k):
        row0 = (chunk0 + k) * _CHK
        pltpu.sync_copy(srcp.at[pl.ds(row0, _CHK), :], buf[0])
        pltpu.sync_copy(dstp.at[pl.ds(row0, _CHK), :], buf[1])

    def fire_gather(buf):
        si, rows, gsem = buf[0], buf[2], buf[3]
        for j in range(_CHK):
            pltpu.async_copy(g.at[si.at[j]], rows.at[pl.ds(j * 128, 128)],
                             gsem)

    def fire_scatter(buf):
        di, rows, ssem = buf[1], buf[2], buf[4]
        for j in range(_CHK):
            pltpu.async_copy(rows.at[pl.ds(j * 128, 128)], acc.at[di.at[j]],
                             ssem, add=True)

    def drain(buf, which):
        pltpu.make_async_copy(g.at[pl.ds(0, _CH)], buf[2], buf[which]).wait()

    pltpu.sync_copy(z16.at[pl.ds(s * _RPT, _RPT)],
                    acc.at[pl.ds(s * _RPT, _RPT)])
    plsc.subcore_barrier()

    load_idx(bufs[0], 0)
    fire_gather(bufs[0])

    def loop(t, carry):
        for b in range(2):
            me, nxt = bufs[b], bufs[1 - b]
            drain(me, 3)
            fire_scatter(me)
            if b == 0:
                @pl.when(t > 0)
                def _():
                    drain(nxt, 4)
            else:
                drain(nxt, 4)
            load_idx(nxt, 2 * t + b + 1)
            fire_gather(nxt)
        return carry

    lax.fori_loop(0, nhalf, loop, 0)
    drain(bufs[1], 4)
    drain(bufs[0], 3)
    plsc.subcore_barrier()

    @pl.when(s == 0)
    def _():
        pltpu.sync_copy(acc, out.at[c])


@functools.partial(
    pl.kernel,
    out_type=jax.ShapeDtypeStruct((2, _NP, 16), jnp.float32),
    mesh=_mesh,
    compiler_params=pltpu.CompilerParams(use_tc_tiling_on_sc=False),
    scratch_types=[
        pltpu.VMEM((_CHK, 128), jnp.int32),
        pltpu.VMEM((_CHK, 128), jnp.int32),
        pltpu.VMEM((_CH, 16), jnp.float32),
        pltpu.VMEM((_CHK, 128), jnp.int32),
        pltpu.VMEM((_CHK, 128), jnp.int32),
        pltpu.VMEM((_CH, 16), jnp.float32),
        pltpu.VMEM_SHARED((_NP, 16), jnp.float32),
        pltpu.SemaphoreType.DMA,
        pltpu.SemaphoreType.DMA,
        pltpu.SemaphoreType.DMA,
        pltpu.SemaphoreType.DMA,
    ],
)
def _layer_sc(srcp, dstp, g, z16, out,
              si0, di0, r0, si1, di1, r1, acc, gs0, gs1, ss0, ss1):
    _layer_body(srcp, dstp, g, z16, out,
                si0, di0, r0, si1, di1, r1, acc, gs0, gs1, ss0, ss1)


def _cnn_body(x_ref, deg16_ref, w1_ref, b1_ref, w2_ref, b2_ref, gw_ref,
              g_ref, dinv_ref):
    x = x_ref[...]
    B = x.shape[0]
    zcol = jnp.zeros((B, 1), jnp.float32)
    xl = jnp.concatenate([zcol, x[:, :-1]], 1)
    xr = jnp.concatenate([x[:, 1:], zcol], 1)
    h1 = []
    for o in range(HIDDEN):
        a = w1_ref[o, 0, 0] * xl + w1_ref[o, 0, 1] * x + w1_ref[o, 0, 2] * xr
        h1.append(jnp.maximum(a + b1_ref[o], 0.0))
    h1l = [jnp.concatenate([zcol, h[:, :-1]], 1) for h in h1]
    h1r = [jnp.concatenate([h[:, 1:], zcol], 1) for h in h1]
    outs = []
    for o in range(HIDDEN):
        acc = jnp.zeros((B, INPUT_DIM), jnp.float32)
        for i in range(HIDDEN):
            acc = acc + w2_ref[o, i, 0] * h1l[i]
            acc = acc + w2_ref[o, i, 1] * h1[i]
            acc = acc + w2_ref[o, i, 2] * h1r[i]
        acc = jnp.maximum(acc + b2_ref[o], 0.0)
        outs.append(jnp.max(acc, axis=1))
    h = jnp.stack(outs, axis=1)
    dinv16 = lax.rsqrt(deg16_ref[...] + 1.0)
    g_ref[...] = jnp.dot(h, gw_ref[...],
                         preferred_element_type=jnp.float32) * dinv16
    dinv_ref[...] = dinv16


def _cnn(x, deg16, conv1_w, conv1_b, conv2_w, conv2_b, gcn1_w):
    grid = (N // _BN,)
    return pl.pallas_call(
        _cnn_body,
        grid=grid,
        in_specs=[
            pl.BlockSpec((_BN, INPUT_DIM), lambda i: (i, 0)),
            pl.BlockSpec((_BN, HIDDEN), lambda i: (i, 0)),
            pl.BlockSpec(memory_space=pltpu.SMEM),
            pl.BlockSpec(memory_space=pltpu.SMEM),
            pl.BlockSpec(memory_space=pltpu.SMEM),
            pl.BlockSpec(memory_space=pltpu.SMEM),
            pl.BlockSpec((HIDDEN, HIDDEN), lambda i: (0, 0)),
        ],
        out_specs=[
            pl.BlockSpec((_BN, HIDDEN), lambda i: (i, 0)),
            pl.BlockSpec((_BN, HIDDEN), lambda i: (i, 0)),
        ],
        out_shape=[
            jax.ShapeDtypeStruct((N, HIDDEN), jnp.float32),
            jax.ShapeDtypeStruct((N, HIDDEN), jnp.float32),
        ],
    )(x, deg16, conv1_w, conv1_b, conv2_w, conv2_b, gcn1_w)


def _mid_body(s0_ref, s1_ref, g1_ref, dinv_ref, b1_ref, w2_ref, g2_ref):
    dinv = dinv_ref[...]
    t = dinv * (s0_ref[0] + s1_ref[0] + g1_ref[...]) + b1_ref[...]
    h2 = jnp.maximum(t, 0.0)
    g2_ref[...] = jnp.dot(h2, w2_ref[...],
                          preferred_element_type=jnp.float32) * dinv


def _mid(s_part, g1, dinv, gcn1_b, gcn2_w):
    grid = (N // _BN,)
    return pl.pallas_call(
        _mid_body,
        grid=grid,
        in_specs=[
            pl.BlockSpec((1, _BN, HIDDEN), lambda i: (0, i, 0)),
            pl.BlockSpec((1, _BN, HIDDEN), lambda i: (1, i, 0)),
            pl.BlockSpec((_BN, HIDDEN), lambda i: (i, 0)),
            pl.BlockSpec((_BN, HIDDEN), lambda i: (i, 0)),
            pl.BlockSpec((1, HIDDEN), lambda i: (0, 0)),
            pl.BlockSpec((HIDDEN, OUT), lambda i: (0, 0)),
        ],
        out_specs=pl.BlockSpec((_BN, OUT), lambda i: (i, 0)),
        out_shape=jax.ShapeDtypeStruct((N, OUT), jnp.float32),
    )(s_part, s_part, g1, dinv, gcn1_b.reshape(1, HIDDEN), gcn2_w)


def _fin_body(s0_ref, s1_ref, g2_ref, dinv_ref, b2_ref, o_ref):
    dinv = dinv_ref[...]
    t = dinv * (s0_ref[0] + s1_ref[0] + g2_ref[...]) + b2_ref[...]
    m = jnp.max(t, axis=1, keepdims=True)
    lse = jnp.log(jnp.sum(jnp.exp(t - m), axis=1, keepdims=True))
    o_ref[...] = t - m - lse


def _fin(s_part, g2, dinv, gcn2_b):
    grid = (N // _BN,)
    return pl.pallas_call(
        _fin_body,
        grid=grid,
        in_specs=[
            pl.BlockSpec((1, _BN, OUT), lambda i: (0, i, 0)),
            pl.BlockSpec((1, _BN, OUT), lambda i: (1, i, 0)),
            pl.BlockSpec((_BN, OUT), lambda i: (i, 0)),
            pl.BlockSpec((_BN, OUT), lambda i: (i, 0)),
            pl.BlockSpec((1, OUT), lambda i: (0, 0)),
        ],
        out_specs=pl.BlockSpec((_BN, OUT), lambda i: (i, 0)),
        out_shape=jax.ShapeDtypeStruct((N, OUT), jnp.float32),
    )(s_part, s_part, g2, dinv, gcn2_b.reshape(1, OUT))


def kernel(x, edge_index, conv1_w, conv1_b, conv2_w, conv2_b,
           gcn1_w, gcn1_b, gcn2_w, gcn2_b):
    pad = _EP + _CH - E
    src_p = jnp.concatenate(
        [edge_index[0], jnp.zeros((pad,), jnp.int32)]).reshape(-1, 128)
    dst_p = jnp.concatenate(
        [edge_index[1], jnp.full((pad,), N, jnp.int32)]).reshape(-1, 128)
    z16 = jnp.zeros((_NP, HIDDEN), jnp.float32)

    degp = _deg_sc(dst_p)
    degsum = (degp[0] + degp[1])[:N]
    deg16 = jnp.broadcast_to(degsum[:, None], (N, HIDDEN))

    g1, dinv = _cnn(x, deg16, conv1_w, conv1_b, conv2_w, conv2_b, gcn1_w)

    s1 = _layer_sc(src_p, dst_p, g1, z16)
    s1 = s1[:, :N, :]

    g2 = _mid(s1, g1, dinv, gcn1_b, gcn2_w)

    s2 = _layer_sc(src_p, dst_p, g2, z16)
    s2 = s2[:, :N, :]

    return _fin(s2, g2, dinv, gcn2_b)


# speedup vs baseline: 1.0367x; 1.0367x over previous
import functools

import jax
import jax.numpy as jnp
from jax import lax
from jax.experimental import pallas as pl
from jax.experimental.pallas import tpu as pltpu
from jax.experimental.pallas import tpu_sc as plsc

N = 50000
E = 3200000
INPUT_DIM = 128
HIDDEN = 16
OUT = 16

_BN = 200
_NW = 32
_NP = 50048
_EPW = 102400
_EP = _EPW * _NW
_CHK = 8
_CH = _CHK * 128
_NCHUNK = _EPW // _CH
_NC0 = 172
_NC1 = 2 * _NCHUNK - _NC0
_RPT = _NP // 16
_NHS = 51200
_DPT = _NHS // 16

_mesh = plsc.VectorSubcoreMesh(core_axis_name="c", subcore_axis_name="s")


def _deg_body(dstp, out, idx_v, ones_v, rid_v, buf_v, acc, sem):
    c = lax.axis_index("c")
    s = lax.axis_index("s")
    wid = s * 2 + c
    for j in range(8):
        ones_v[pl.ds(j * 16, 16)] = jnp.ones((16,), jnp.float32)
        buf_v[pl.ds(j * 16, 16)] = jnp.zeros((16,), jnp.float32)
    base = s * _DPT
    for k in range(_DPT // 128):
        for o in range(8):
            rid_v[k, pl.ds(o * 16, 16)] = (
                jnp.arange(16, dtype=jnp.int32) + (base + k * 128 + o * 16))
    zcps = [pltpu.async_copy(buf_v.at[pl.ds(0, 128)], acc.at[rid_v.at[k]], sem)
            for k in range(_DPT // 128)]
    for cp in zcps:
        cp.wait()
    plsc.subcore_barrier()

    def body(i, carry):
        row0 = wid * (_EPW // 128) + i * _CHK
        pltpu.sync_copy(dstp.at[pl.ds(row0, _CHK), :], idx_v)
        cps = [pltpu.async_copy(ones_v, acc.at[idx_v.at[j]], sem, add=True)
               for j in range(_CHK)]
        for cp in cps:
            cp.wait()
        return carry

    lax.fori_loop(0, _NCHUNK, body, 0)
    plsc.subcore_barrier()
    gcps = [pltpu.async_copy(acc.at[rid_v.at[k]],
                             buf_v.at[pl.ds(k * 128, 128)], sem)
            for k in range(_DPT // 128)]
    for cp in gcps:
        cp.wait()
    pltpu.sync_copy(buf_v, out.at[c, pl.ds(base, _DPT)])


@functools.partial(
    pl.kernel,
    out_type=jax.ShapeDtypeStruct((2, _NHS), jnp.float32),
    mesh=_mesh,
    scratch_types=[
        pltpu.VMEM((_CHK, 128), jnp.int32),
        pltpu.VMEM((128,), jnp.float32),
        pltpu.VMEM((_DPT // 128, 128), jnp.int32),
        pltpu.VMEM((_DPT,), jnp.float32),
        pltpu.VMEM_SHARED((_NHS,), jnp.float32),
        pltpu.SemaphoreType.DMA,
    ],
)
def _deg_sc(dstp, out, idx_v, ones_v, rid_v, buf_v, acc, sem):
    _deg_body(dstp, out, idx_v, ones_v, rid_v, buf_v, acc, sem)


def _layer_body(srcp, dstp, g, z16, out,
                si0, di0, r0, si1, di1, r1, acc,
                gs0, gs1, ss0, ss1):
    c = lax.axis_index("c")
    s = lax.axis_index("s")
    bufs = ((si0, di0, r0, gs0, ss0), (si1, di1, r1, gs1, ss1))
    chunk0 = jnp.where(c == 0, s * _NC0, 16 * _NC0 + s * _NC1)
    nhalf = jnp.where(c == 0, _NC0 // 2, _NC1 // 2)

    def load_idx(buf, k):
        row0 = (chunk0 + k) * _CHK
        pltpu.sync_copy(srcp.at[pl.ds(row0, _CHK), :], buf[0])
        pltpu.sync_copy(dstp.at[pl.ds(row0, _CHK), :], buf[1])

    def fire_gather(buf):
        si, rows, gsem = buf[0], buf[2], buf[3]
        for j in range(_CHK):
            pltpu.async_copy(g.at[si.at[j]], rows.at[pl.ds(j * 128, 128)],
                             gsem)

    def fire_scatter(buf):
        di, rows, ssem = buf[1], buf[2], buf[4]
        for j in range(_CHK):
            pltpu.async_copy(rows.at[pl.ds(j * 128, 128)], acc.at[di.at[j]],
                             ssem, add=True)

    def drain(buf, which):
        pltpu.make_async_copy(g.at[pl.ds(0, _CH)], buf[2], buf[which]).wait()

    pltpu.sync_copy(z16.at[pl.ds(s * _RPT, _RPT)],
                    acc.at[pl.ds(s * _RPT, _RPT)])
    plsc.subcore_barrier()

    load_idx(bufs[0], 0)
    fire_gather(bufs[0])

    def loop(t, carry):
        for b in range(2):
            me, nxt = bufs[b], bufs[1 - b]
            drain(me, 3)
            fire_scatter(me)
            if b == 0:
                @pl.when(t > 0)
                def _():
                    drain(nxt, 4)
            else:
                drain(nxt, 4)
            load_idx(nxt, 2 * t + b + 1)
            fire_gather(nxt)
        return carry

    lax.fori_loop(0, nhalf, loop, 0)
    drain(bufs[1], 4)
    drain(bufs[0], 3)
    plsc.subcore_barrier()

    @pl.when(s == 0)
    def _():
        pltpu.sync_copy(acc, out.at[c])


@functools.partial(
    pl.kernel,
    out_type=jax.ShapeDtypeStruct((2, _NP, 16), jnp.float32),
    mesh=_mesh,
    compiler_params=pltpu.CompilerParams(use_tc_tiling_on_sc=False),
    scratch_types=[
        pltpu.VMEM((_CHK, 128), jnp.int32),
        pltpu.VMEM((_CHK, 128), jnp.int32),
        pltpu.VMEM((_CH, 16), jnp.float32),
        pltpu.VMEM((_CHK, 128), jnp.int32),
        pltpu.VMEM((_CHK, 128), jnp.int32),
        pltpu.VMEM((_CH, 16), jnp.float32),
        pltpu.VMEM_SHARED((_NP, 16), jnp.float32),
        pltpu.SemaphoreType.DMA,
        pltpu.SemaphoreType.DMA,
        pltpu.SemaphoreType.DMA,
        pltpu.SemaphoreType.DMA,
    ],
)
def _layer_sc(srcp, dstp, g, z16, out,
              si0, di0, r0, si1, di1, r1, acc, gs0, gs1, ss0, ss1):
    _layer_body(srcp, dstp, g, z16, out,
                si0, di0, r0, si1, di1, r1, acc, gs0, gs1, ss0, ss1)


def _cnn_body(x_ref, deg16_ref, w1_ref, b1_ref, w2_ref, b2_ref, gw_ref,
              g_ref, dinv_ref):
    x = x_ref[...]
    B = x.shape[0]
    zcol = jnp.zeros((B, 1), jnp.float32)
    xl = jnp.concatenate([zcol, x[:, :-1]], 1)
    xr = jnp.concatenate([x[:, 1:], zcol], 1)
    h1 = []
    for o in range(HIDDEN):
        a = w1_ref[o, 0, 0] * xl + w1_ref[o, 0, 1] * x + w1_ref[o, 0, 2] * xr
        h1.append(jnp.maximum(a + b1_ref[o], 0.0))
    h1l = [jnp.concatenate([zcol, h[:, :-1]], 1) for h in h1]
    h1r = [jnp.concatenate([h[:, 1:], zcol], 1) for h in h1]
    outs = []
    for o in range(HIDDEN):
        acc = jnp.zeros((B, INPUT_DIM), jnp.float32)
        for i in range(HIDDEN):
            acc = acc + w2_ref[o, i, 0] * h1l[i]
            acc = acc + w2_ref[o, i, 1] * h1[i]
            acc = acc + w2_ref[o, i, 2] * h1r[i]
        acc = jnp.maximum(acc + b2_ref[o], 0.0)
        outs.append(jnp.max(acc, axis=1))
    h = jnp.stack(outs, axis=1)
    dinv16 = lax.rsqrt(deg16_ref[...] + 1.0)
    g_ref[...] = jnp.dot(h, gw_ref[...],
                         preferred_element_type=jnp.float32) * dinv16
    dinv_ref[...] = dinv16


def _cnn(x, deg16, conv1_w, conv1_b, conv2_w, conv2_b, gcn1_w):
    grid = (N // _BN,)
    return pl.pallas_call(
        _cnn_body,
        grid=grid,
        in_specs=[
            pl.BlockSpec((_BN, INPUT_DIM), lambda i: (i, 0)),
            pl.BlockSpec((_BN, HIDDEN), lambda i: (i, 0)),
            pl.BlockSpec(memory_space=pltpu.SMEM),
            pl.BlockSpec(memory_space=pltpu.SMEM),
            pl.BlockSpec(memory_space=pltpu.SMEM),
            pl.BlockSpec(memory_space=pltpu.SMEM),
            pl.BlockSpec((HIDDEN, HIDDEN), lambda i: (0, 0)),
        ],
        out_specs=[
            pl.BlockSpec((_BN, HIDDEN), lambda i: (i, 0)),
            pl.BlockSpec((_BN, HIDDEN), lambda i: (i, 0)),
        ],
        out_shape=[
            jax.ShapeDtypeStruct((N, HIDDEN), jnp.float32),
            jax.ShapeDtypeStruct((N, HIDDEN), jnp.float32),
        ],
    )(x, deg16, conv1_w, conv1_b, conv2_w, conv2_b, gcn1_w)


def _mid_body(s0_ref, s1_ref, g1_ref, dinv_ref, b1_ref, w2_ref, g2_ref):
    dinv = dinv_ref[...]
    t = dinv * (s0_ref[0] + s1_ref[0] + g1_ref[...]) + b1_ref[...]
    h2 = jnp.maximum(t, 0.0)
    g2_ref[...] = jnp.dot(h2, w2_ref[...],
                          preferred_element_type=jnp.float32) * dinv


def _mid(s_part, g1, dinv, gcn1_b, gcn2_w):
    grid = (N // _BN,)
    return pl.pallas_call(
        _mid_body,
        grid=grid,
        in_specs=[
            pl.BlockSpec((1, _BN, HIDDEN), lambda i: (0, i, 0)),
            pl.BlockSpec((1, _BN, HIDDEN), lambda i: (1, i, 0)),
            pl.BlockSpec((_BN, HIDDEN), lambda i: (i, 0)),
            pl.BlockSpec((_BN, HIDDEN), lambda i: (i, 0)),
            pl.BlockSpec((1, HIDDEN), lambda i: (0, 0)),
            pl.BlockSpec((HIDDEN, OUT), lambda i: (0, 0)),
        ],
        out_specs=pl.BlockSpec((_BN, OUT), lambda i: (i, 0)),
        out_shape=jax.ShapeDtypeStruct((N, OUT), jnp.float32),
    )(s_part, s_part, g1, dinv, gcn1_b.reshape(1, HIDDEN), gcn2_w)


def _fin_body(s0_ref, s1_ref, g2_ref, dinv_ref, b2_ref, o_ref):
    dinv = dinv_ref[...]
    t = dinv * (s0_ref[0] + s1_ref[0] + g2_ref[...]) + b2_ref[...]
    m = jnp.max(t, axis=1, keepdims=True)
    lse = jnp.log(jnp.sum(jnp.exp(t - m), axis=1, keepdims=True))
    o_ref[...] = t - m - lse


def _fin(s_part, g2, dinv, gcn2_b):
    grid = (N // _BN,)
    return pl.pallas_call(
        _fin_body,
        grid=grid,
        in_specs=[
            pl.BlockSpec((1, _BN, OUT), lambda i: (0, i, 0)),
            pl.BlockSpec((1, _BN, OUT), lambda i: (1, i, 0)),
            pl.BlockSpec((_BN, OUT), lambda i: (i, 0)),
            pl.BlockSpec((_BN, OUT), lambda i: (i, 0)),
            pl.BlockSpec((1, OUT), lambda i: (0, 0)),
        ],
        out_specs=pl.BlockSpec((_BN, OUT), lambda i: (i, 0)),
        out_shape=jax.ShapeDtypeStruct((N, OUT), jnp.float32),
    )(s_part, s_part, g2, dinv, gcn2_b.reshape(1, OUT))


def kernel(x, edge_index, conv1_w, conv1_b, conv2_w, conv2_b,
           gcn1_w, gcn1_b, gcn2_w, gcn2_b):
    pad = _EP + _CH - E
    src_p = jnp.concatenate(
        [edge_index[0], jnp.zeros((pad,), jnp.int32)]).reshape(-1, 128)
    dst_p = jnp.concatenate(
        [edge_index[1], jnp.full((pad,), N, jnp.int32)]).reshape(-1, 128)
    z16 = jnp.zeros((_NP, HIDDEN), jnp.float32)

    degp = _deg_sc(dst_p)
    degsum = (degp[0] + degp[1])[:N]
    deg16 = jnp.broadcast_to(degsum[:, None], (N, HIDDEN))

    g1, dinv = _cnn(x, deg16, conv1_w, conv1_b, conv2_w, conv2_b, gcn1_w)

    s1 = _layer_sc(src_p, dst_p, g1, z16)
    s1 = s1[:, :N, :]

    g2 = _mid(s1, g1, dinv, gcn1_b, gcn2_w)

    s2 = _layer_sc(src_p, dst_p, g2, z16)
    s2 = s2[:, :N, :]

    return _fin(s2, g2, dinv, gcn2_b)


# speedup vs baseline: 1.0412x; 1.0043x over previous
import functools

import jax
import jax.numpy as jnp
from jax import lax
from jax.experimental import pallas as pl
from jax.experimental.pallas import tpu as pltpu
from jax.experimental.pallas import tpu_sc as plsc

N = 50000
E = 3200000
INPUT_DIM = 128
HIDDEN = 16
OUT = 16

_BN = 200
_NW = 32
_NP = 50048
_EPW = 102400
_EP = _EPW * _NW
_CHK = 8
_CH = _CHK * 128
_NCHUNK = _EPW // _CH
_NC0 = 168
_NC1 = 2 * _NCHUNK - _NC0
_RPT = _NP // 16
_NHS = 51200
_DPT = _NHS // 16

_mesh = plsc.VectorSubcoreMesh(core_axis_name="c", subcore_axis_name="s")


def _deg_body(dstp, out, idx_v, ones_v, rid_v, buf_v, acc, sem):
    c = lax.axis_index("c")
    s = lax.axis_index("s")
    wid = s * 2 + c
    for j in range(8):
        ones_v[pl.ds(j * 16, 16)] = jnp.ones((16,), jnp.float32)
        buf_v[pl.ds(j * 16, 16)] = jnp.zeros((16,), jnp.float32)
    base = s * _DPT
    for k in range(_DPT // 128):
        for o in range(8):
            rid_v[k, pl.ds(o * 16, 16)] = (
                jnp.arange(16, dtype=jnp.int32) + (base + k * 128 + o * 16))
    zcps = [pltpu.async_copy(buf_v.at[pl.ds(0, 128)], acc.at[rid_v.at[k]], sem)
            for k in range(_DPT // 128)]
    for cp in zcps:
        cp.wait()
    plsc.subcore_barrier()

    def body(i, carry):
        row0 = wid * (_EPW // 128) + i * _CHK
        pltpu.sync_copy(dstp.at[pl.ds(row0, _CHK), :], idx_v)
        cps = [pltpu.async_copy(ones_v, acc.at[idx_v.at[j]], sem, add=True)
               for j in range(_CHK)]
        for cp in cps:
            cp.wait()
        return carry

    lax.fori_loop(0, _NCHUNK, body, 0)
    plsc.subcore_barrier()
    gcps = [pltpu.async_copy(acc.at[rid_v.at[k]],
                             buf_v.at[pl.ds(k * 128, 128)], sem)
            for k in range(_DPT // 128)]
    for cp in gcps:
        cp.wait()
    pltpu.sync_copy(buf_v, out.at[c, pl.ds(base, _DPT)])


@functools.partial(
    pl.kernel,
    out_type=jax.ShapeDtypeStruct((2, _NHS), jnp.float32),
    mesh=_mesh,
    scratch_types=[
        pltpu.VMEM((_CHK, 128), jnp.int32),
        pltpu.VMEM((128,), jnp.float32),
        pltpu.VMEM((_DPT // 128, 128), jnp.int32),
        pltpu.VMEM((_DPT,), jnp.float32),
        pltpu.VMEM_SHARED((_NHS,), jnp.float32),
        pltpu.SemaphoreType.DMA,
    ],
)
def _deg_sc(dstp, out, idx_v, ones_v, rid_v, buf_v, acc, sem):
    _deg_body(dstp, out, idx_v, ones_v, rid_v, buf_v, acc, sem)


def _layer_body(srcp, dstp, g, z16, out,
                si0, di0, r0, si1, di1, r1, acc,
                gs0, gs1, ss0, ss1):
    c = lax.axis_index("c")
    s = lax.axis_index("s")
    bufs = ((si0, di0, r0, gs0, ss0), (si1, di1, r1, gs1, ss1))
    chunk0 = jnp.where(c == 0, s * _NC0, 16 * _NC0 + s * _NC1)
    nhalf = jnp.where(c == 0, _NC0 // 2, _NC1 // 2)

    def load_idx(buf, k):
        row0 = (chunk0 + k) * _CHK
        pltpu.sync_copy(srcp.at[pl.ds(row0, _CHK), :], buf[0])
        pltpu.sync_copy(dstp.at[pl.ds(row0, _CHK), :], buf[1])

    def fire_gather(buf):
        si, rows, gsem = buf[0], buf[2], buf[3]
        for j in range(_CHK):
            pltpu.async_copy(g.at[si.at[j]], rows.at[pl.ds(j * 128, 128)],
                             gsem)

    def fire_scatter(buf):
        di, rows, ssem = buf[1], buf[2], buf[4]
        for j in range(_CHK):
            pltpu.async_copy(rows.at[pl.ds(j * 128, 128)], acc.at[di.at[j]],
                             ssem, add=True)

    def drain(buf, which):
        pltpu.make_async_copy(g.at[pl.ds(0, _CH)], buf[2], buf[which]).wait()

    pltpu.sync_copy(z16.at[pl.ds(s * _RPT, _RPT)],
                    acc.at[pl.ds(s * _RPT, _RPT)])
    plsc.subcore_barrier()

    load_idx(bufs[0], 0)
    fire_gather(bufs[0])

    def loop(t, carry):
        for b in range(2):
            me, nxt = bufs[b], bufs[1 - b]
            drain(me, 3)
            fire_scatter(me)
            if b == 0:
                @pl.when(t > 0)
                def _():
                    drain(nxt, 4)
            else:
                drain(nxt, 4)
            load_idx(nxt, 2 * t + b + 1)
            fire_gather(nxt)
        return carry

    lax.fori_loop(0, nhalf, loop, 0)
    drain(bufs[1], 4)
    drain(bufs[0], 3)
    plsc.subcore_barrier()

    @pl.when(s == 0)
    def _():
        pltpu.sync_copy(acc, out.at[c])


@functools.partial(
    pl.kernel,
    out_type=jax.ShapeDtypeStruct((2, _NP, 16), jnp.float32),
    mesh=_mesh,
    compiler_params=pltpu.CompilerParams(use_tc_tiling_on_sc=False),
    scratch_types=[
        pltpu.VMEM((_CHK, 128), jnp.int32),
        pltpu.VMEM((_CHK, 128), jnp.int32),
        pltpu.VMEM((_CH, 16), jnp.float32),
        pltpu.VMEM((_CHK, 128), jnp.int32),
        pltpu.VMEM((_CHK, 128), jnp.int32),
        pltpu.VMEM((_CH, 16), jnp.float32),
        pltpu.VMEM_SHARED((_NP, 16), jnp.float32),
        pltpu.SemaphoreType.DMA,
        pltpu.SemaphoreType.DMA,
        pltpu.SemaphoreType.DMA,
        pltpu.SemaphoreType.DMA,
    ],
)
def _layer_sc(srcp, dstp, g, z16, out,
              si0, di0, r0, si1, di1, r1, acc, gs0, gs1, ss0, ss1):
    _layer_body(srcp, dstp, g, z16, out,
                si0, di0, r0, si1, di1, r1, acc, gs0, gs1, ss0, ss1)


def _cnn_body(x_ref, deg16_ref, w1_ref, b1_ref, w2_ref, b2_ref, gw_ref,
              g_ref, dinv_ref):
    x = x_ref[...]
    B = x.shape[0]
    zcol = jnp.zeros((B, 1), jnp.float32)
    xl = jnp.concatenate([zcol, x[:, :-1]], 1)
    xr = jnp.concatenate([x[:, 1:], zcol], 1)
    h1 = []
    for o in range(HIDDEN):
        a = w1_ref[o, 0, 0] * xl + w1_ref[o, 0, 1] * x + w1_ref[o, 0, 2] * xr
        h1.append(jnp.maximum(a + b1_ref[o], 0.0))
    h1l = [jnp.concatenate([zcol, h[:, :-1]], 1) for h in h1]
    h1r = [jnp.concatenate([h[:, 1:], zcol], 1) for h in h1]
    outs = []
    for o in range(HIDDEN):
        acc = jnp.zeros((B, INPUT_DIM), jnp.float32)
        for i in range(HIDDEN):
            acc = acc + w2_ref[o, i, 0] * h1l[i]
            acc = acc + w2_ref[o, i, 1] * h1[i]
            acc = acc + w2_ref[o, i, 2] * h1r[i]
        acc = jnp.maximum(acc + b2_ref[o], 0.0)
        outs.append(jnp.max(acc, axis=1))
    h = jnp.stack(outs, axis=1)
    dinv16 = lax.rsqrt(deg16_ref[...] + 1.0)
    g_ref[...] = jnp.dot(h, gw_ref[...],
                         preferred_element_type=jnp.float32) * dinv16
    dinv_ref[...] = dinv16


def _cnn(x, deg16, conv1_w, conv1_b, conv2_w, conv2_b, gcn1_w):
    grid = (N // _BN,)
    return pl.pallas_call(
        _cnn_body,
        grid=grid,
        in_specs=[
            pl.BlockSpec((_BN, INPUT_DIM), lambda i: (i, 0)),
            pl.BlockSpec((_BN, HIDDEN), lambda i: (i, 0)),
            pl.BlockSpec(memory_space=pltpu.SMEM),
            pl.BlockSpec(memory_space=pltpu.SMEM),
            pl.BlockSpec(memory_space=pltpu.SMEM),
            pl.BlockSpec(memory_space=pltpu.SMEM),
            pl.BlockSpec((HIDDEN, HIDDEN), lambda i: (0, 0)),
        ],
        out_specs=[
            pl.BlockSpec((_BN, HIDDEN), lambda i: (i, 0)),
            pl.BlockSpec((_BN, HIDDEN), lambda i: (i, 0)),
        ],
        out_shape=[
            jax.ShapeDtypeStruct((N, HIDDEN), jnp.float32),
            jax.ShapeDtypeStruct((N, HIDDEN), jnp.float32),
        ],
    )(x, deg16, conv1_w, conv1_b, conv2_w, conv2_b, gcn1_w)


def _mid_body(s0_ref, s1_ref, g1_ref, dinv_ref, b1_ref, w2_ref, g2_ref):
    dinv = dinv_ref[...]
    t = dinv * (s0_ref[0] + s1_ref[0] + g1_ref[...]) + b1_ref[...]
    h2 = jnp.maximum(t, 0.0)
    g2_ref[...] = jnp.dot(h2, w2_ref[...],
                          preferred_element_type=jnp.float32) * dinv


def _mid(s_part, g1, dinv, gcn1_b, gcn2_w):
    grid = (N // _BN,)
    return pl.pallas_call(
        _mid_body,
        grid=grid,
        in_specs=[
            pl.BlockSpec((1, _BN, HIDDEN), lambda i: (0, i, 0)),
            pl.BlockSpec((1, _BN, HIDDEN), lambda i: (1, i, 0)),
            pl.BlockSpec((_BN, HIDDEN), lambda i: (i, 0)),
            pl.BlockSpec((_BN, HIDDEN), lambda i: (i, 0)),
            pl.BlockSpec((1, HIDDEN), lambda i: (0, 0)),
            pl.BlockSpec((HIDDEN, OUT), lambda i: (0, 0)),
        ],
        out_specs=pl.BlockSpec((_BN, OUT), lambda i: (i, 0)),
        out_shape=jax.ShapeDtypeStruct((N, OUT), jnp.float32),
    )(s_part, s_part, g1, dinv, gcn1_b.reshape(1, HIDDEN), gcn2_w)


def _fin_body(s0_ref, s1_ref, g2_ref, dinv_ref, b2_ref, o_ref):
    dinv = dinv_ref[...]
    t = dinv * (s0_ref[0] + s1_ref[0] + g2_ref[...]) + b2_ref[...]
    m = jnp.max(t, axis=1, keepdims=True)
    lse = jnp.log(jnp.sum(jnp.exp(t - m), axis=1, keepdims=True))
    o_ref[...] = t - m - lse


def _fin(s_part, g2, dinv, gcn2_b):
    grid = (N // _BN,)
    return pl.pallas_call(
        _fin_body,
        grid=grid,
        in_specs=[
            pl.BlockSpec((1, _BN, OUT), lambda i: (0, i, 0)),
            pl.BlockSpec((1, _BN, OUT), lambda i: (1, i, 0)),
            pl.BlockSpec((_BN, OUT), lambda i: (i, 0)),
            pl.BlockSpec((_BN, OUT), lambda i: (i, 0)),
            pl.BlockSpec((1, OUT), lambda i: (0, 0)),
        ],
        out_specs=pl.BlockSpec((_BN, OUT), lambda i: (i, 0)),
        out_shape=jax.ShapeDtypeStruct((N, OUT), jnp.float32),
    )(s_part, s_part, g2, dinv, gcn2_b.reshape(1, OUT))


def kernel(x, edge_index, conv1_w, conv1_b, conv2_w, conv2_b,
           gcn1_w, gcn1_b, gcn2_w, gcn2_b):
    pad = _EP + _CH - E
    src_p = jnp.concatenate(
        [edge_index[0], jnp.zeros((pad,), jnp.int32)]).reshape(-1, 128)
    dst_p = jnp.concatenate(
        [edge_index[1], jnp.full((pad,), N, jnp.int32)]).reshape(-1, 128)
    z16 = jnp.zeros((_NP, HIDDEN), jnp.float32)

    degp = _deg_sc(dst_p)
    degsum = (degp[0] + degp[1])[:N]
    deg16 = jnp.broadcast_to(degsum[:, None], (N, HIDDEN))

    g1, dinv = _cnn(x, deg16, conv1_w, conv1_b, conv2_w, conv2_b, gcn1_w)

    s1 = _layer_sc(src_p, dst_p, g1, z16)
    s1 = s1[:, :N, :]

    g2 = _mid(s1, g1, dinv, gcn1_b, gcn2_w)

    s2 = _layer_sc(src_p, dst_p, g2, z16)
    s2 = s2[:, :N, :]

    return _fin(s2, g2, dinv, gcn2_b)


# speedup vs baseline: 1.5425x; 1.4815x over previous
import functools

import jax
import jax.numpy as jnp
from jax import lax
from jax.experimental import pallas as pl
from jax.experimental.pallas import tpu as pltpu
from jax.experimental.pallas import tpu_sc as plsc

N = 50000
E = 3200000
INPUT_DIM = 128
HIDDEN = 16
OUT = 16

_BN = 200
_NW = 32
_NP = 50048
_EPW = 102400
_EP = _EPW * _NW
_CHK = 8
_CH = _CHK * 128
_NCHUNK = _EPW // _CH
_NC0 = 168
_NC1 = 2 * _NCHUNK - _NC0
_RPT = _NP // 16
_NHS = 51200
_DPT = _NHS // 16

_mesh = plsc.VectorSubcoreMesh(core_axis_name="c", subcore_axis_name="s")


def _deg_body(dstp, out, idx_v, ones_v, rid_v, buf_v, acc, sem):
    c = lax.axis_index("c")
    s = lax.axis_index("s")
    wid = s * 2 + c
    for j in range(8):
        ones_v[pl.ds(j * 16, 16)] = jnp.ones((16,), jnp.float32)
        buf_v[pl.ds(j * 16, 16)] = jnp.zeros((16,), jnp.float32)
    base = s * _DPT
    for k in range(_DPT // 128):
        for o in range(8):
            rid_v[k, pl.ds(o * 16, 16)] = (
                jnp.arange(16, dtype=jnp.int32) + (base + k * 128 + o * 16))
    zcps = [pltpu.async_copy(buf_v.at[pl.ds(0, 128)], acc.at[rid_v.at[k]], sem)
            for k in range(_DPT // 128)]
    for cp in zcps:
        cp.wait()
    plsc.subcore_barrier()

    def body(i, carry):
        row0 = wid * (_EPW // 128) + i * _CHK
        pltpu.sync_copy(dstp.at[pl.ds(row0, _CHK), :], idx_v)
        cps = [pltpu.async_copy(ones_v, acc.at[idx_v.at[j]], sem, add=True)
               for j in range(_CHK)]
        for cp in cps:
            cp.wait()
        return carry

    lax.fori_loop(0, _NCHUNK, body, 0)
    plsc.subcore_barrier()
    gcps = [pltpu.async_copy(acc.at[rid_v.at[k]],
                             buf_v.at[pl.ds(k * 128, 128)], sem)
            for k in range(_DPT // 128)]
    for cp in gcps:
        cp.wait()
    pltpu.sync_copy(buf_v, out.at[c, pl.ds(base, _DPT)])


@functools.partial(
    pl.kernel,
    out_type=jax.ShapeDtypeStruct((2, _NHS), jnp.float32),
    mesh=_mesh,
    scratch_types=[
        pltpu.VMEM((_CHK, 128), jnp.int32),
        pltpu.VMEM((128,), jnp.float32),
        pltpu.VMEM((_DPT // 128, 128), jnp.int32),
        pltpu.VMEM((_DPT,), jnp.float32),
        pltpu.VMEM_SHARED((_NHS,), jnp.float32),
        pltpu.SemaphoreType.DMA,
    ],
)
def _deg_sc(dstp, out, idx_v, ones_v, rid_v, buf_v, acc, sem):
    _deg_body(dstp, out, idx_v, ones_v, rid_v, buf_v, acc, sem)


def _layer_body(srcp, dstp, g, z16, out,
                si0, di0, r0, si1, di1, r1, acc,
                gs0, gs1, ss0, ss1):
    c = lax.axis_index("c")
    s = lax.axis_index("s")
    bufs = ((si0, di0, r0, gs0, ss0), (si1, di1, r1, gs1, ss1))
    chunk0 = jnp.where(c == 0, s * _NC0, 16 * _NC0 + s * _NC1)
    nhalf = jnp.where(c == 0, _NC0 // 2, _NC1 // 2)

    def load_idx(buf, k):
        row0 = (chunk0 + k) * _CHK
        pltpu.sync_copy(srcp.at[pl.ds(row0, _CHK), :], buf[0])
        pltpu.sync_copy(dstp.at[pl.ds(row0, _CHK), :], buf[1])

    def fire_gather(buf):
        si, rows, gsem = buf[0], buf[2], buf[3]
        for j in range(_CHK):
            pltpu.async_copy(g.at[si.at[j]], rows.at[pl.ds(j * 128, 128)],
                             gsem)

    def fire_scatter(buf):
        di, rows, ssem = buf[1], buf[2], buf[4]
        for j in range(_CHK):
            pltpu.async_copy(rows.at[pl.ds(j * 128, 128)], acc.at[di.at[j]],
                             ssem, add=True)

    def drain(buf, which):
        pltpu.make_async_copy(g.at[pl.ds(0, _CH)], buf[2], buf[which]).wait()

    pltpu.sync_copy(z16.at[pl.ds(s * _RPT, _RPT)],
                    acc.at[pl.ds(s * _RPT, _RPT)])
    plsc.subcore_barrier()

    load_idx(bufs[0], 0)
    fire_gather(bufs[0])

    def loop(t, carry):
        for b in range(2):
            me, nxt = bufs[b], bufs[1 - b]
            drain(me, 3)
            fire_scatter(me)
            if b == 0:
                @pl.when(t > 0)
                def _():
                    drain(nxt, 4)
            else:
                drain(nxt, 4)
            load_idx(nxt, 2 * t + b + 1)
            fire_gather(nxt)
        return carry

    lax.fori_loop(0, nhalf, loop, 0)
    drain(bufs[1], 4)
    drain(bufs[0], 3)
    plsc.subcore_barrier()

    @pl.when(s == 0)
    def _():
        pltpu.sync_copy(acc, out.at[c])


@functools.partial(
    pl.kernel,
    out_type=jax.ShapeDtypeStruct((2, _NP, 16), jnp.float32),
    mesh=_mesh,
    compiler_params=pltpu.CompilerParams(use_tc_tiling_on_sc=False),
    scratch_types=[
        pltpu.VMEM((_CHK, 128), jnp.int32),
        pltpu.VMEM((_CHK, 128), jnp.int32),
        pltpu.VMEM((_CH, 16), jnp.float32),
        pltpu.VMEM((_CHK, 128), jnp.int32),
        pltpu.VMEM((_CHK, 128), jnp.int32),
        pltpu.VMEM((_CH, 16), jnp.float32),
        pltpu.VMEM_SHARED((_NP, 16), jnp.float32),
        pltpu.SemaphoreType.DMA,
        pltpu.SemaphoreType.DMA,
        pltpu.SemaphoreType.DMA,
        pltpu.SemaphoreType.DMA,
    ],
)
def _layer_sc(srcp, dstp, g, z16, out,
              si0, di0, r0, si1, di1, r1, acc, gs0, gs1, ss0, ss1):
    _layer_body(srcp, dstp, g, z16, out,
                si0, di0, r0, si1, di1, r1, acc, gs0, gs1, ss0, ss1)


def _cnn_body(x_ref, deg16_ref, wband_ref, w1_ref, b1_ref, b2_ref, gw_ref,
              g_ref, dinv_ref):
    x = x_ref[...]
    B = x.shape[0]
    zcol = jnp.zeros((B, 1), jnp.float32)
    xl = jnp.concatenate([zcol, x[:, :-1]], 1)
    xr = jnp.concatenate([x[:, 1:], zcol], 1)
    h1 = []
    for o in range(HIDDEN):
        a = w1_ref[o, 0, 0] * xl + w1_ref[o, 0, 1] * x + w1_ref[o, 0, 2] * xr
        h1.append(jnp.maximum(a + b1_ref[o], 0.0))
    hflat = jnp.concatenate(h1, axis=1)
    conv2 = jnp.dot(hflat.astype(jnp.bfloat16), wband_ref[...],
                    preferred_element_type=jnp.float32)
    outs = []
    for o in range(HIDDEN):
        acc = conv2[:, o * INPUT_DIM:(o + 1) * INPUT_DIM] + b2_ref[o]
        acc = jnp.maximum(acc, 0.0)
        outs.append(jnp.max(acc, axis=1))
    h = jnp.stack(outs, axis=1)
    dinv16 = lax.rsqrt(deg16_ref[...] + 1.0)
    g_ref[...] = jnp.dot(h, gw_ref[...],
                         preferred_element_type=jnp.float32) * dinv16
    dinv_ref[...] = dinv16


def _cnn(x, deg16, wband, conv1_w, conv1_b, conv2_b, gcn1_w):
    grid = (N // _BN,)
    return pl.pallas_call(
        _cnn_body,
        grid=grid,
        in_specs=[
            pl.BlockSpec((_BN, INPUT_DIM), lambda i: (i, 0)),
            pl.BlockSpec((_BN, HIDDEN), lambda i: (i, 0)),
            pl.BlockSpec((HIDDEN * INPUT_DIM, HIDDEN * INPUT_DIM),
                         lambda i: (0, 0)),
            pl.BlockSpec(memory_space=pltpu.SMEM),
            pl.BlockSpec(memory_space=pltpu.SMEM),
            pl.BlockSpec(memory_space=pltpu.SMEM),
            pl.BlockSpec((HIDDEN, HIDDEN), lambda i: (0, 0)),
        ],
        out_specs=[
            pl.BlockSpec((_BN, HIDDEN), lambda i: (i, 0)),
            pl.BlockSpec((_BN, HIDDEN), lambda i: (i, 0)),
        ],
        out_shape=[
            jax.ShapeDtypeStruct((N, HIDDEN), jnp.float32),
            jax.ShapeDtypeStruct((N, HIDDEN), jnp.float32),
        ],
    )(x, deg16, wband, conv1_w, conv1_b, conv2_b, gcn1_w)


def _mid_body(s0_ref, s1_ref, g1_ref, dinv_ref, b1_ref, w2_ref, g2_ref):
    dinv = dinv_ref[...]
    t = dinv * (s0_ref[0] + s1_ref[0] + g1_ref[...]) + b1_ref[...]
    h2 = jnp.maximum(t, 0.0)
    g2_ref[...] = jnp.dot(h2, w2_ref[...],
                          preferred_element_type=jnp.float32) * dinv


def _mid(s_part, g1, dinv, gcn1_b, gcn2_w):
    grid = (N // _BN,)
    return pl.pallas_call(
        _mid_body,
        grid=grid,
        in_specs=[
            pl.BlockSpec((1, _BN, HIDDEN), lambda i: (0, i, 0)),
            pl.BlockSpec((1, _BN, HIDDEN), lambda i: (1, i, 0)),
            pl.BlockSpec((_BN, HIDDEN), lambda i: (i, 0)),
            pl.BlockSpec((_BN, HIDDEN), lambda i: (i, 0)),
            pl.BlockSpec((1, HIDDEN), lambda i: (0, 0)),
            pl.BlockSpec((HIDDEN, OUT), lambda i: (0, 0)),
        ],
        out_specs=pl.BlockSpec((_BN, OUT), lambda i: (i, 0)),
        out_shape=jax.ShapeDtypeStruct((N, OUT), jnp.float32),
    )(s_part, s_part, g1, dinv, gcn1_b.reshape(1, HIDDEN), gcn2_w)


def _fin_body(s0_ref, s1_ref, g2_ref, dinv_ref, b2_ref, o_ref):
    dinv = dinv_ref[...]
    t = dinv * (s0_ref[0] + s1_ref[0] + g2_ref[...]) + b2_ref[...]
    m = jnp.max(t, axis=1, keepdims=True)
    lse = jnp.log(jnp.sum(jnp.exp(t - m), axis=1, keepdims=True))
    o_ref[...] = t - m - lse


def _fin(s_part, g2, dinv, gcn2_b):
    grid = (N // _BN,)
    return pl.pallas_call(
        _fin_body,
        grid=grid,
        in_specs=[
            pl.BlockSpec((1, _BN, OUT), lambda i: (0, i, 0)),
            pl.BlockSpec((1, _BN, OUT), lambda i: (1, i, 0)),
            pl.BlockSpec((_BN, OUT), lambda i: (i, 0)),
            pl.BlockSpec((_BN, OUT), lambda i: (i, 0)),
            pl.BlockSpec((1, OUT), lambda i: (0, 0)),
        ],
        out_specs=pl.BlockSpec((_BN, OUT), lambda i: (i, 0)),
        out_shape=jax.ShapeDtypeStruct((N, OUT), jnp.float32),
    )(s_part, s_part, g2, dinv, gcn2_b.reshape(1, OUT))


def kernel(x, edge_index, conv1_w, conv1_b, conv2_w, conv2_b,
           gcn1_w, gcn1_b, gcn2_w, gcn2_b):
    pad = _EP + _CH - E
    src_p = jnp.concatenate(
        [edge_index[0], jnp.zeros((pad,), jnp.int32)]).reshape(-1, 128)
    dst_p = jnp.concatenate(
        [edge_index[1], jnp.full((pad,), N, jnp.int32)]).reshape(-1, 128)
    z16 = jnp.zeros((_NP, HIDDEN), jnp.float32)

    degp = _deg_sc(dst_p)
    degsum = (degp[0] + degp[1])[:N]
    deg16 = jnp.broadcast_to(degsum[:, None], (N, HIDDEN))

    eyes = [jnp.eye(INPUT_DIM, k=1 - k, dtype=jnp.float32) for k in range(3)]
    wband = sum(jnp.kron(conv2_w[:, :, k].T, eyes[k]) for k in range(3))
    wband = wband.astype(jnp.bfloat16)

    g1, dinv = _cnn(x, deg16, wband, conv1_w, conv1_b, conv2_b, gcn1_w)

    s1 = _layer_sc(src_p, dst_p, g1, z16)
    s1 = s1[:, :N, :]

    g2 = _mid(s1, g1, dinv, gcn1_b, gcn2_w)

    s2 = _layer_sc(src_p, dst_p, g2, z16)
    s2 = s2[:, :N, :]

    return _fin(s2, g2, dinv, gcn2_b)
